# baseline (speedup 1.0000x reference)
import functools

import jax
import jax.numpy as jnp
import numpy as np
from jax import lax
from jax.experimental import pallas as pl
from jax.experimental.pallas import tpu as pltpu
from jax.experimental.pallas import tpu_sc as plsc

_LIMS_F32 = np.linspace(-np.float32(np.pi), np.float32(np.pi), 20).astype(np.float32)
_COS_T = [np.float64(np.cos(np.float64(_LIMS_F32[k]))) for k in range(10, 19)]
_POS_T2 = [float(np.float32(t * t)) for t in _COS_T if t > 0]
_NEG_T2 = [float(np.float32(t * t)) for t in _COS_T if t < 0]

_NW = 32


def _sc_body(x_hbm, idx_hbm, o_hbm, inbuf, idxbuf, outbuf, addrbuf, sem, *, bsz, n, rpw):
    wid = lax.axis_index("s") * 2 + lax.axis_index("c")
    lanes = lax.broadcasted_iota(jnp.int32, (16,), 0)
    ones = jnp.full((16,), 1.0, dtype=jnp.float32)
    zerosv = jnp.zeros((16,), dtype=jnp.float32)
    lastlane = n - 1
    ngrp = n // 16

    def _zero(j, c):
        outbuf[pl.ds(j * 16, 16)] = zerosv
        return c

    lax.fori_loop(0, 63 * n // 16, _zero, 0)

    def bin_of(dot, q):
        rho = (dot * dot) / q
        neg = dot < 0.0
        acc = jnp.full((16,), 10, dtype=jnp.int32)
        for t2 in _POS_T2:
            acc = acc + jnp.where(neg | (rho < t2), 1, 0)
        for t2 in _NEG_T2:
            acc = acc + jnp.where(neg & (rho > t2), 1, 0)
        return acc

    def cosparts(a, bv, g):
        v1 = [a[i] - bv[i] for i in range(3)]
        v2 = [g[i] - bv[i] for i in range(3)]
        dot = v1[0] * v2[0] + v1[1] * v2[1] + v1[2] * v2[2]
        s1 = v1[0] * v1[0] + v1[1] * v1[1] + v1[2] * v1[2]
        s2 = v2[0] * v2[0] + v2[1] * v2[1] + v2[2] * v2[2]
        return dot, s1 * s2

    for rb in range(rpw):
        brow = wid * rpw + rb
        for p in range(9):
            pltpu.sync_copy(
                x_hbm.at[pl.ds((p * bsz + brow) * n, n)],
                inbuf.at[pl.ds(p * n, n)],
            )
        pltpu.sync_copy(idx_hbm.at[pl.ds(brow * n, n)], idxbuf)

        def group(g, c):
            r0 = g * 16
            rows = r0 + lanes

            def comp(p):
                return inbuf[pl.ds(p * n + r0, 16)]

            def compn(p):
                return plsc.load_gather(inbuf, [rows + (p * n + 1)])

            na = [comp(0), comp(1), comp(2)]
            ca = [comp(3), comp(4), comp(5)]
            cc = [comp(6), comp(7), comp(8)]
            nan_ = [compn(0), compn(1), compn(2)]
            can_ = [compn(3), compn(4), compn(5)]

            d1, q1 = cosparts(na, ca, cc)
            d2, q2 = cosparts(ca, cc, nan_)
            d3, q3 = cosparts(cc, nan_, can_)

            idxv = idxbuf[pl.ds(r0, 16)]
            idxn = plsc.load_gather(idxbuf, [rows + 1])
            good = ((idxn - idxv) == 1) & (rows != lastlane)

            b1 = bin_of(d1, q1)
            b2 = jnp.where(good, bin_of(d2, q2), 10)
            b3 = jnp.where(good, bin_of(d3, q3), 10)

            a1 = b1 * n + rows
            a2 = (b2 + 21) * n + rows
            a3 = (b3 + 42) * n + rows
            plsc.store_scatter(outbuf, [a1], ones)
            plsc.store_scatter(outbuf, [a2], ones)
            plsc.store_scatter(outbuf, [a3], ones)
            abase = g * 48
            addrbuf[pl.ds(abase, 16)] = a1
            addrbuf[pl.ds(abase + 16, 16)] = a2
            addrbuf[pl.ds(abase + 32, 16)] = a3
            return c

        lax.fori_loop(0, ngrp, group, 0)

        copies = []
        for p in range(63):
            copies.append(
                pltpu.make_async_copy(
                    outbuf.at[pl.ds(p * n, n)],
                    o_hbm.at[pl.ds((p * bsz + brow) * n, n)],
                    sem,
                )
            )
        for cp in copies:
            cp.start()
        for cp in copies:
            cp.wait()

        def clean(j, c):
            addr = addrbuf[pl.ds(j * 16, 16)]
            plsc.store_scatter(outbuf, [addr], zerosv)
            return c

        lax.fori_loop(0, 3 * ngrp, clean, 0)


def kernel(coords, mask, residue_pdb_idx):
    del mask
    b, n = coords.shape[0], coords.shape[1]
    assert b % _NW == 0 and n % 16 == 0
    rpw = b // _NW
    nat3 = coords.shape[2] * coords.shape[3]

    xt = jnp.transpose(coords, (2, 3, 0, 1)).reshape(nat3, b, n)[:9].reshape(9 * b * n)
    idxflat = residue_pdb_idx.astype(jnp.int32).reshape(b * n)

    mesh = plsc.VectorSubcoreMesh(core_axis_name="c", subcore_axis_name="s")
    run = pl.kernel(
        functools.partial(_sc_body, bsz=b, n=n, rpw=rpw),
        mesh=mesh,
        compiler_params=pltpu.CompilerParams(needs_layout_passes=False),
        out_type=jax.ShapeDtypeStruct((63 * b * n,), jnp.float32),
        scratch_types=[
            pltpu.VMEM((9 * n,), jnp.float32),
            pltpu.VMEM((n,), jnp.int32),
            pltpu.VMEM((63 * n,), jnp.float32),
            pltpu.VMEM((3 * n,), jnp.int32),
            pltpu.SemaphoreType.DMA,
        ],
    )
    out = run(xt, idxflat)
    return out.reshape(63, b, n).transpose(1, 2, 0)


# speedup vs baseline: 1.4316x; 1.4316x over previous
import functools

import jax
import jax.numpy as jnp
import numpy as np
from jax import lax
from jax.experimental import pallas as pl
from jax.experimental.pallas import tpu as pltpu
from jax.experimental.pallas import tpu_sc as plsc

_LIMS_F32 = np.linspace(-np.float32(np.pi), np.float32(np.pi), 20).astype(np.float32)
_COS_T = [np.float64(np.cos(np.float64(_LIMS_F32[k]))) for k in range(10, 19)]
_POS_T2 = [float(np.float32(t * t)) for t in _COS_T if t > 0]
_NEG_T2 = [float(np.float32(t * t)) for t in _COS_T if t < 0]

_NW = 32


def _sc_body(x_hbm, idx_hbm, o_hbm, inbuf, idxbuf, outbuf, addrbuf, sem, insem, *, bsz, n, rpw):
    wid = lax.axis_index("s") * 2 + lax.axis_index("c")
    lanes = lax.broadcasted_iota(jnp.int32, (16,), 0)
    ones = jnp.full((16,), 1.0, dtype=jnp.float32)
    zerosv = jnp.zeros((16,), dtype=jnp.float32)
    lastlane = n - 1
    ngrp = n // 16
    npad = n + 16

    def stage(rb, half):
        brow = wid * rpw + rb
        cps = [
            pltpu.make_async_copy(
                x_hbm.at[pl.ds((p * bsz + brow) * n, n)],
                inbuf.at[pl.ds((half * 9 + p) * n, n)],
                insem,
            )
            for p in range(9)
        ]
        cps.append(
            pltpu.make_async_copy(
                idx_hbm.at[pl.ds(brow * n, n)],
                idxbuf.at[pl.ds(half * npad, n)],
                insem,
            )
        )
        for cp in cps:
            cp.start()
        return cps

    pend = stage(0, 0)

    def _zero(j, c):
        outbuf[pl.ds(j * 64, 16)] = zerosv
        outbuf[pl.ds(j * 64 + 16, 16)] = zerosv
        outbuf[pl.ds(j * 64 + 32, 16)] = zerosv
        outbuf[pl.ds(j * 64 + 48, 16)] = zerosv
        return c

    lax.fori_loop(0, 63 * n // 64, _zero, 0)

    def bin_of(dot, q):
        rho = (dot * dot) / q
        neg = dot < 0.0
        acc = jnp.full((16,), 10, dtype=jnp.int32)
        for t2 in _POS_T2:
            acc = acc + jnp.where(neg | (rho < t2), 1, 0)
        for t2 in _NEG_T2:
            acc = acc + jnp.where(neg & (rho > t2), 1, 0)
        return acc

    def cosparts(a, bv, g):
        v1 = [a[i] - bv[i] for i in range(3)]
        v2 = [g[i] - bv[i] for i in range(3)]
        dot = v1[0] * v2[0] + v1[1] * v2[1] + v1[2] * v2[2]
        s1 = v1[0] * v1[0] + v1[1] * v1[1] + v1[2] * v1[2]
        s2 = v2[0] * v2[0] + v2[1] * v2[1] + v2[2] * v2[2]
        return dot, s1 * s2

    for rb in range(rpw):
        brow = wid * rpw + rb
        half = rb % 2
        for cp in pend:
            cp.wait()
        if rb + 1 < rpw:
            pend = stage(rb + 1, 1 - half)

        def group(g, c):
            r0 = g * 16
            rows = r0 + lanes
            ib = half * 9 * n
            jb = half * npad

            def comp(p):
                return inbuf[pl.ds(ib + p * n + r0, 16)]

            def compn(p):
                return inbuf[pl.ds(ib + p * n + r0 + 1, 16)]

            na = [comp(0), comp(1), comp(2)]
            ca = [comp(3), comp(4), comp(5)]
            cc = [comp(6), comp(7), comp(8)]
            nan_ = [compn(0), compn(1), compn(2)]
            can_ = [compn(3), compn(4), compn(5)]

            d1, q1 = cosparts(na, ca, cc)
            d2, q2 = cosparts(ca, cc, nan_)
            d3, q3 = cosparts(cc, nan_, can_)

            idxv = idxbuf[pl.ds(jb + r0, 16)]
            idxn = idxbuf[pl.ds(jb + r0 + 1, 16)]
            good = ((idxn - idxv) == 1) & (rows != lastlane)

            b1 = bin_of(d1, q1)
            b2 = jnp.where(good, bin_of(d2, q2), 10)
            b3 = jnp.where(good, bin_of(d3, q3), 10)

            a1 = b1 * n + rows
            a2 = (b2 + 21) * n + rows
            a3 = (b3 + 42) * n + rows
            plsc.store_scatter(outbuf, [a1], ones)
            plsc.store_scatter(outbuf, [a2], ones)
            plsc.store_scatter(outbuf, [a3], ones)
            abase = g * 48
            addrbuf[pl.ds(abase, 16)] = a1
            addrbuf[pl.ds(abase + 16, 16)] = a2
            addrbuf[pl.ds(abase + 32, 16)] = a3
            return c

        lax.fori_loop(0, ngrp, group, 0)

        copies = []
        for p in range(63):
            copies.append(
                pltpu.make_async_copy(
                    outbuf.at[pl.ds(p * n, n)],
                    o_hbm.at[pl.ds((p * bsz + brow) * n, n)],
                    sem,
                )
            )
        for cp in copies:
            cp.start()
        for cp in copies:
            cp.wait()

        def clean(j, c):
            addr = addrbuf[pl.ds(j * 16, 16)]
            plsc.store_scatter(outbuf, [addr], zerosv)
            return c

        lax.fori_loop(0, 3 * ngrp, clean, 0)


def kernel(coords, mask, residue_pdb_idx):
    del mask
    b, n = coords.shape[0], coords.shape[1]
    assert b % _NW == 0 and n % 16 == 0
    rpw = b // _NW
    nat3 = coords.shape[2] * coords.shape[3]

    xt = jnp.transpose(coords, (2, 3, 0, 1)).reshape(nat3, b, n)[:9].reshape(9 * b * n)
    idxflat = residue_pdb_idx.astype(jnp.int32).reshape(b * n)

    mesh = plsc.VectorSubcoreMesh(core_axis_name="c", subcore_axis_name="s")
    run = pl.kernel(
        functools.partial(_sc_body, bsz=b, n=n, rpw=rpw),
        mesh=mesh,
        compiler_params=pltpu.CompilerParams(needs_layout_passes=False),
        out_type=jax.ShapeDtypeStruct((63 * b * n,), jnp.float32),
        scratch_types=[
            pltpu.VMEM((2 * 9 * n,), jnp.float32),
            pltpu.VMEM((2 * (n + 16),), jnp.int32),
            pltpu.VMEM((63 * n,), jnp.float32),
            pltpu.VMEM((3 * n,), jnp.int32),
            pltpu.SemaphoreType.DMA,
            pltpu.SemaphoreType.DMA,
        ],
    )
    out = run(xt, idxflat)
    return out.reshape(63, b, n).transpose(1, 2, 0)
